# baseline (speedup 1.0000x reference)
import functools
import numpy as np
import jax
import jax.numpy as jnp
from jax import lax
from jax.experimental import pallas as pl
from jax.experimental.pallas import tpu as pltpu
from jax.experimental.pallas import tpu_sc as plsc

DT = 0.01
MAX_T = 2.0
STEPS = 200
N = 100000
NWORK = 25
TPW = 3968
TAIL = N - NWORK * TPW
NCHUNK = STEPS // 8
BIG = 1e9


def _sc_kernel(x_hbm, scal_hbm, noise_hbm, pr_hbm, pc_hbm,
               scal_v, xbuf, nzbuf, dv_v, mu_v, ml_v, ds_v, pr_v, pc_v):
    wid = lax.axis_index("s") * 2 + lax.axis_index("c")

    def run(base, tpw):
        ng = tpw // 16
        pltpu.sync_copy(scal_hbm, scal_v)
        pltpu.sync_copy(x_hbm.at[pl.ds(base, tpw)], xbuf.at[pl.ds(0, tpw)])

        av = scal_v[0, :]
        zv = scal_v[1, :]
        ndtv = scal_v[2, :]
        gv = scal_v[3, :]
        th_hi = av - 1e-6 - zv * av
        th_lo = -av + 1e-6 - zv * av
        big16 = jnp.full((16,), BIG, jnp.float32)

        def init_g(g, _):
            sl = pl.ds(g * 16, 16)
            dv_v[sl] = zv * av
            mu_v[sl] = big16
            ml_v[sl] = big16
            ds_v[sl] = gv * DT * xbuf[sl]
            return _
        lax.fori_loop(0, ng, init_g, None)

        def chunk(tc, _):
            for st in range(8):
                pltpu.sync_copy(
                    noise_hbm.at[pl.ds((tc * 8 + st) * N + base, tpw)],
                    nzbuf.at[pl.ds(st * TPW, tpw)])
            t0 = jnp.full((16,), 8.0, jnp.float32) * tc.astype(jnp.float32)

            def group(g, _):
                sl = pl.ds(g * 16, 16)
                d16 = ds_v[sl]
                dv = dv_v[sl]
                mu = mu_v[sl]
                ml = ml_v[sl]
                for st in range(8):
                    nz = nzbuf[pl.ds(st * TPW + g * 16, 16)]
                    dv = dv + d16 + nz
                    tv = t0 + np.float32(st)
                    mu = jnp.minimum(mu, jnp.where(dv >= th_hi, tv, big16))
                    ml = jnp.minimum(ml, jnp.where(dv <= th_lo, tv, big16))
                dv_v[sl] = dv
                mu_v[sl] = mu
                ml_v[sl] = ml
                return _
            lax.fori_loop(0, ng, group, None)
            return _
        lax.fori_loop(0, NCHUNK, chunk, None)

        def fin(g, _):
            sl = pl.ds(g * 16, 16)
            mu = mu_v[sl]
            ml = ml_v[sl]
            tf = jnp.minimum(mu, ml)
            hit = tf < big16
            pr_v[sl] = jnp.where(hit, tf * DT + ndtv, MAX_T + ndtv)
            pc_v[sl] = jnp.where(
                hit, jnp.where(mu <= ml, 1.0, 0.0), 0.5)
            return _
        lax.fori_loop(0, ng, fin, None)

        pltpu.sync_copy(pr_v.at[pl.ds(0, tpw)], pr_hbm.at[pl.ds(base, tpw)])
        pltpu.sync_copy(pc_v.at[pl.ds(0, tpw)], pc_hbm.at[pl.ds(base, tpw)])

    @pl.when(wid < NWORK)
    def _():
        run(wid * TPW, TPW)

    @pl.when(wid == NWORK)
    def _():
        run(NWORK * TPW, TAIL)


@jax.jit
def kernel(x, a, z, ndt, drift_gain, noise):
    xf = x.reshape(-1)
    scal = jnp.stack([jnp.full((16,), v, jnp.float32)
                      for v in (a, z, ndt, drift_gain)])
    run = functools.partial(
        pl.kernel,
        mesh=plsc.VectorSubcoreMesh(core_axis_name="c", subcore_axis_name="s"),
        out_type=[
            jax.ShapeDtypeStruct((N,), jnp.float32),
            jax.ShapeDtypeStruct((N,), jnp.float32),
        ],
        scratch_types=[
            pltpu.VMEM((4, 16), jnp.float32),
            pltpu.VMEM((TPW,), jnp.float32),
            pltpu.VMEM((8 * TPW,), jnp.float32),
            pltpu.VMEM((TPW,), jnp.float32),
            pltpu.VMEM((TPW,), jnp.float32),
            pltpu.VMEM((TPW,), jnp.float32),
            pltpu.VMEM((TPW,), jnp.float32),
            pltpu.VMEM((TPW,), jnp.float32),
            pltpu.VMEM((TPW,), jnp.float32),
        ],
    )(_sc_kernel)
    pr, pc = run(xf, scal, noise.reshape(-1))
    return pr, pc


# speedup vs baseline: 3.0621x; 3.0621x over previous
import numpy as np
import jax
import jax.numpy as jnp
from jax.experimental import pallas as pl
from jax.experimental.pallas import tpu as pltpu

DT = 0.01
MAX_T = 2.0
STEPS = 200
SC = 100
SP = 112
NSC = STEPS // SC
BN = 4096
BIGF = 1e9


def _ddm_block(x_ref, w_ref, a_ref, z_ref, ndt_ref, g_ref, noise_ref,
               pr_ref, pc_ref, xc_ref, base_ref, mu_ref, ml_ref):
    j = pl.program_id(1)
    a = a_ref[0, 0]
    z = z_ref[0, 0]
    ndt = ndt_ref[0, 0]
    gain = g_ref[0, 0]

    drift_dt = (gain * DT) * x_ref[...]

    nz = noise_ref[0]
    hi = nz.astype(jnp.bfloat16)
    r1 = nz - hi.astype(jnp.float32)
    mid = r1.astype(jnp.bfloat16)
    lo = (r1 - mid.astype(jnp.float32)).astype(jnp.bfloat16)
    xc_ref[0:SC, :] = hi
    xc_ref[SP:SP + SC, :] = mid
    xc_ref[2 * SP:2 * SP + SC, :] = lo

    d_hi_f = drift_dt.astype(jnp.bfloat16).astype(jnp.float32)
    dr = drift_dt - d_hi_f
    d_mid_f = dr.astype(jnp.bfloat16).astype(jnp.float32)
    d_lo = dr - d_mid_f
    rowp = jax.lax.broadcasted_iota(jnp.int32, (SP - SC, BN), 0)
    m0 = rowp == 0
    xc_ref[SC:SP, :] = jnp.where(m0, drift_dt, 0.0).astype(jnp.bfloat16)
    xc_ref[SP + SC:2 * SP, :] = jnp.where(m0, dr, 0.0).astype(jnp.bfloat16)
    xc_ref[2 * SP + SC:3 * SP, :] = jnp.where(m0, d_lo, 0.0).astype(jnp.bfloat16)

    s = jax.lax.dot(w_ref[...], xc_ref[...],
                    preferred_element_type=jnp.float32)

    zero_row = jnp.zeros((1, BN), jnp.float32)
    base = jnp.where(j == 0, zero_row, base_ref[...])
    th_hi = ((a - 1e-6) - z * a) - base
    th_lo = ((-a + 1e-6) - z * a) - base

    t_f = jax.lax.broadcasted_iota(
        jnp.int32, (SC, BN), 0).astype(jnp.float32)
    enc_u = jnp.where(s >= th_hi, t_f, BIGF)
    enc_l = jnp.where(s <= th_lo, t_f, BIGF)
    off = jnp.float32(SC) * j.astype(jnp.float32)
    mu = jnp.min(enc_u, axis=0, keepdims=True) + off
    ml = jnp.min(enc_l, axis=0, keepdims=True) + off

    @pl.when(j == 0)
    def _():
        base_ref[...] = base + s[SC - 1:SC, :]
        mu_ref[...] = mu
        ml_ref[...] = ml

    @pl.when(j == NSC - 1)
    def _():
        mu_f = jnp.minimum(mu_ref[...], mu)
        ml_f = jnp.minimum(ml_ref[...], ml)
        t_first = jnp.minimum(mu_f, ml_f)
        hit = t_first < BIGF
        pr_ref[...] = jnp.where(hit, t_first * DT + ndt, MAX_T + ndt)
        pc_ref[...] = jnp.where(hit, jnp.where(mu_f <= ml_f, 1.0, 0.0), 0.5)


def _weights():
    tril = np.tril(np.ones((SC, SC), np.float32))
    tcol = np.arange(1, SC + 1, dtype=np.float32).reshape(SC, 1)
    sec = np.concatenate([tril, tcol, np.zeros((SC, SP - SC - 1),
                                               np.float32)], axis=1)
    return jnp.asarray(np.concatenate([sec] * 3, axis=1), dtype=jnp.bfloat16)


@jax.jit
def kernel(x, a, z, ndt, drift_gain, noise):
    n = x.shape[0]
    x2 = x.reshape(1, n)
    noise3 = noise.reshape(NSC, SC, n)
    w = _weights()
    grid = (pl.cdiv(n, BN), NSC)
    scal = pl.BlockSpec(memory_space=pltpu.SMEM)
    pr, pc = pl.pallas_call(
        _ddm_block,
        grid=grid,
        in_specs=[
            pl.BlockSpec((1, BN), lambda i, j: (0, i)),
            pl.BlockSpec((SC, 3 * SP), lambda i, j: (0, 0)),
            scal, scal, scal, scal,
            pl.BlockSpec((1, SC, BN), lambda i, j: (j, 0, i)),
        ],
        out_specs=[
            pl.BlockSpec((1, BN), lambda i, j: (0, i)),
            pl.BlockSpec((1, BN), lambda i, j: (0, i)),
        ],
        out_shape=[
            jax.ShapeDtypeStruct((1, n), jnp.float32),
            jax.ShapeDtypeStruct((1, n), jnp.float32),
        ],
        scratch_shapes=[
            pltpu.VMEM((3 * SP, BN), jnp.bfloat16),
            pltpu.VMEM((1, BN), jnp.float32),
            pltpu.VMEM((1, BN), jnp.float32),
            pltpu.VMEM((1, BN), jnp.float32),
        ],
    )(x2, w,
      a.reshape(1, 1), z.reshape(1, 1), ndt.reshape(1, 1),
      drift_gain.reshape(1, 1), noise3)
    return pr.reshape(n), pc.reshape(n)


# speedup vs baseline: 7.0583x; 2.3050x over previous
import numpy as np
import jax
import jax.numpy as jnp
from jax.experimental import pallas as pl
from jax.experimental.pallas import tpu as pltpu

DT = 0.01
MAX_T = 2.0
STEPS = 200
SP = 208
BN = 5120
BIGF = 1e9


def _ddm_block(x_ref, w_ref, a_ref, z_ref, ndt_ref, g_ref, noise_ref,
               pr_ref, pc_ref, xc_ref):
    a = a_ref[0, 0]
    z = z_ref[0, 0]
    ndt = ndt_ref[0, 0]
    gain = g_ref[0, 0]

    drift_dt = (gain * DT) * x_ref[...]
    th_hi = (a - 1e-6) - z * a
    th_lo = (-a + 1e-6) - z * a

    nz = noise_ref[...]
    hi = nz.astype(jnp.bfloat16)
    r1 = nz - hi.astype(jnp.float32)
    mid = r1.astype(jnp.bfloat16)
    lo = (r1 - mid.astype(jnp.float32)).astype(jnp.bfloat16)
    xc_ref[0:STEPS, :] = hi
    xc_ref[SP:SP + STEPS, :] = mid
    xc_ref[2 * SP:2 * SP + STEPS, :] = lo

    d_hi_f = drift_dt.astype(jnp.bfloat16).astype(jnp.float32)
    dr = drift_dt - d_hi_f
    d_mid_f = dr.astype(jnp.bfloat16).astype(jnp.float32)
    d_lo = dr - d_mid_f
    row8 = jax.lax.broadcasted_iota(jnp.int32, (8, BN), 0)
    m0 = row8 == 0
    xc_ref[STEPS:SP, :] = jnp.where(m0, drift_dt, 0.0).astype(jnp.bfloat16)
    xc_ref[SP + STEPS:2 * SP, :] = jnp.where(m0, dr, 0.0).astype(jnp.bfloat16)
    xc_ref[2 * SP + STEPS:3 * SP, :] = jnp.where(m0, d_lo, 0.0).astype(jnp.bfloat16)

    s = jax.lax.dot(w_ref[...], xc_ref[...],
                    preferred_element_type=jnp.float32)

    t_f = jax.lax.broadcasted_iota(
        jnp.int32, (STEPS, BN), 0).astype(jnp.float32)
    enc_u = jnp.where(s >= th_hi, t_f, BIGF)
    enc_l = jnp.where(s <= th_lo, t_f, BIGF)
    mu = jnp.min(enc_u, axis=0, keepdims=True)
    ml = jnp.min(enc_l, axis=0, keepdims=True)

    t_first = jnp.minimum(mu, ml)
    hit = t_first < BIGF
    pr_ref[...] = jnp.where(hit, t_first * DT + ndt, MAX_T + ndt)
    pc_ref[...] = jnp.where(hit, jnp.where(mu <= ml, 1.0, 0.0), 0.5)


def _weights():
    tril = np.tril(np.ones((STEPS, STEPS), np.float32))
    tcol = np.arange(1, STEPS + 1, dtype=np.float32).reshape(STEPS, 1)
    sec = np.concatenate([tril, tcol, np.zeros((STEPS, SP - STEPS - 1),
                                               np.float32)], axis=1)
    return jnp.asarray(np.concatenate([sec] * 3, axis=1), dtype=jnp.bfloat16)


@jax.jit
def kernel(x, a, z, ndt, drift_gain, noise):
    n = x.shape[0]
    x2 = x.reshape(1, n)
    w = _weights()
    grid = (pl.cdiv(n, BN),)
    scal = pl.BlockSpec(memory_space=pltpu.SMEM)
    pr, pc = pl.pallas_call(
        _ddm_block,
        grid=grid,
        in_specs=[
            pl.BlockSpec((1, BN), lambda i: (0, i)),
            pl.BlockSpec((STEPS, 3 * SP), lambda i: (0, 0)),
            scal, scal, scal, scal,
            pl.BlockSpec((STEPS, BN), lambda i: (0, i)),
        ],
        out_specs=[
            pl.BlockSpec((1, BN), lambda i: (0, i)),
            pl.BlockSpec((1, BN), lambda i: (0, i)),
        ],
        out_shape=[
            jax.ShapeDtypeStruct((1, n), jnp.float32),
            jax.ShapeDtypeStruct((1, n), jnp.float32),
        ],
        scratch_shapes=[pltpu.VMEM((3 * SP, BN), jnp.bfloat16)],
    )(x2, w,
      a.reshape(1, 1), z.reshape(1, 1), ndt.reshape(1, 1),
      drift_gain.reshape(1, 1), noise)
    return pr.reshape(n), pc.reshape(n)
